# baseline (speedup 1.0000x reference)
import functools

import jax
import jax.numpy as jnp
from jax import lax
from jax.experimental import pallas as pl
from jax.experimental.pallas import tpu as pltpu
from jax.experimental.pallas import tpu_sc as plsc

_LANES = 16
_CHUNK = 8192
_IDXW = 128
_ROWS = _CHUNK // _IDXW
_ALIGN = 8 * _IDXW


def _sc_body(ns, total_supp, seg_w,
             ctrlp, jm2, jlow, tp, cpad, out,
             shtab, cseg, idx2d, jlv, tpv, rows, acc,
             sem_i, sem_t, sem_g):
  iota = lax.iota(jnp.int32, _LANES)
  m3 = iota < 3
  fzero = jnp.zeros((_LANES,), jnp.float32)
  max_row0 = total_supp // _IDXW - _ROWS

  sid = lax.axis_index("s")
  wid = lax.axis_index("c") * ns + sid
  s0 = wid * seg_w

  @pl.when(sid == 0)
  def _():
    pltpu.sync_copy(ctrlp, shtab)

  plsc.subcore_barrier()

  pltpu.sync_copy(cpad.at[pl.ds(s0, seg_w + 32)], cseg)

  def _cs(i):
    return cseg[pl.ds(i, _LANES)][0]

  p1 = _cs(seg_w)

  def seg_body(s_, wbase):
    cs = _cs(s_)
    cn = _cs(s_ + 1)
    plen = cn - cs
    nsteps = (plen + (_LANES - 1)) // _LANES

    def step(i, carry):
      wb, vx, vy, vz = carry
      bs = cs + i * _LANES
      need_end = jnp.minimum(bs + _LANES, p1)
      refill = need_end > wb + _CHUNK
      new_base = bs & ~jnp.int32(_ALIGN - 1)
      row0 = jnp.minimum(new_base // _IDXW, max_row0)
      row0 = pl.multiple_of(row0, 8)
      wb_new = jnp.where(refill, row0 * _IDXW, wb)

      @pl.when(refill)
      def _():
        cp_i = pltpu.async_copy(jm2.at[pl.ds(row0, _ROWS)], idx2d, sem_i)
        cp_t = pltpu.async_copy(tp.at[pl.ds(row0 * _IDXW, _CHUNK)], tpv,
                                sem_t)
        cp_l = pltpu.async_copy(jlow.at[pl.ds(row0 * _IDXW, _CHUNK)], jlv,
                                sem_t)
        cp_i.wait()

        def fire(j, _):
          pltpu.async_copy(shtab.at[idx2d.at[j]],
                           rows.at[pl.ds(j * _IDXW, _IDXW)], sem_g)
          return 0

        lax.fori_loop(0, _ROWS, fire, 0)
        cp_t.wait()
        cp_l.wait()
        pltpu.make_async_copy(ctrlp.at[idx2d.at[0]], rows, sem_g).wait()

      off = (bs - wb_new) + iota
      valid = (i * _LANES + iota) < plen
      tpg = plsc.load_gather(tpv, [off], mask=valid)
      tpm = jnp.where(valid, tpg, 0.0)
      c0 = plsc.load_gather(jlv, [off], mask=valid) * 3
      gx = plsc.load_gather(rows, [off, c0], mask=valid)
      gy = plsc.load_gather(rows, [off, c0 + 1], mask=valid)
      gz = plsc.load_gather(rows, [off, c0 + 2], mask=valid)
      vx = vx + tpm * jnp.where(valid, gx, 0.0)
      vy = vy + tpm * jnp.where(valid, gy, 0.0)
      vz = vz + tpm * jnp.where(valid, gz, 0.0)
      return (wb_new, vx, vy, vz)

    wbase, vx, vy, vz = lax.fori_loop(0, nsteps, step,
                                      (wbase, fzero, fzero, fzero))
    sx = jnp.sum(vx)
    sy = jnp.sum(vy)
    sz = jnp.sum(vz)
    contrib = (jnp.where(iota == 0, sx, 0.0)
               + jnp.where(iota == 1, sy, 0.0)
               + jnp.where(iota == 2, sz, 0.0))
    plsc.store_scatter(acc, [4 * s_ + iota], contrib, mask=m3)
    return wbase

  lax.fori_loop(0, seg_w, seg_body, jnp.int32(-(2 ** 30)))

  pltpu.sync_copy(acc.at[pl.ds(0, seg_w * 4)],
                  out.at[pl.ds(wid * seg_w * 4, seg_w * 4)])


def kernel(ctrl_pts, Jm_array, tensor_prod, num_supp_bs_cumsum):
  num_ctrl = ctrl_pts.shape[0]
  total_supp = Jm_array.shape[0]
  num_eval = num_supp_bs_cumsum.shape[0] - 1

  try:
    info = plsc.get_sparse_core_info()
    nc, ns = info.num_cores, info.num_subcores
  except ValueError:
    nc, ns = 2, 16
  nw = nc * ns
  seg_w = num_eval // nw
  assert num_eval % nw == 0 and total_supp % _CHUNK == 0
  assert num_ctrl % 2 == 0

  ctrlp = jnp.pad(ctrl_pts.reshape(num_ctrl // 2, 6), ((0, 0), (0, 2)))
  jm2 = (Jm_array >> 1).reshape(total_supp // _IDXW, _IDXW)
  jlow = Jm_array & 1
  cpad = jnp.pad(num_supp_bs_cumsum, (0, 32))

  accw = seg_w * 4 + _LANES

  mesh = plsc.VectorSubcoreMesh(core_axis_name="c", subcore_axis_name="s",
                                num_cores=nc, num_subcores=ns)
  out_flat = pl.kernel(
      functools.partial(_sc_body, ns, total_supp, seg_w),
      out_type=jax.ShapeDtypeStruct((num_eval * 4,), jnp.float32),
      mesh=mesh,
      compiler_params=pltpu.CompilerParams(needs_layout_passes=False,
                                           use_tc_tiling_on_sc=False),
      scratch_types=[
          pltpu.VMEM_SHARED((num_ctrl // 2, 8), jnp.float32),
          pltpu.VMEM((seg_w + 32,), jnp.int32),
          pltpu.VMEM((_ROWS, _IDXW), jnp.int32),
          pltpu.VMEM((_CHUNK,), jnp.int32),
          pltpu.VMEM((_CHUNK,), jnp.float32),
          pltpu.VMEM((_CHUNK, 8), jnp.float32),
          pltpu.VMEM((accw,), jnp.float32),
          pltpu.SemaphoreType.DMA,
          pltpu.SemaphoreType.DMA,
          pltpu.SemaphoreType.DMA,
      ],
  )(ctrlp, jm2, jlow, tensor_prod, cpad)

  return out_flat.reshape(num_eval, 4)[:, :3]


# speedup vs baseline: 1.2153x; 1.2153x over previous
import functools

import jax
import jax.numpy as jnp
from jax import lax
from jax.experimental import pallas as pl
from jax.experimental.pallas import tpu as pltpu
from jax.experimental.pallas import tpu_sc as plsc

_LANES = 16
_CHUNK = 4096
_IDXW = 128
_ROWS = _CHUNK // _IDXW
_ALIGN = 8 * _IDXW


def _sc_body(ns, total_supp, seg_w,
             ctrl8, jm2, tp, cpad, out,
             shtab, cseg, idx2d, tpv, rows, acc,
             sem_i, sem_t, sem_g):
  iota = lax.iota(jnp.int32, _LANES)
  col1 = jnp.full((_LANES,), 1, jnp.int32)
  col2 = jnp.full((_LANES,), 2, jnp.int32)
  col0 = jnp.zeros((_LANES,), jnp.int32)
  m3 = iota < 3
  fzero = jnp.zeros((_LANES,), jnp.float32)
  max_row0 = total_supp // _IDXW - _ROWS

  sid = lax.axis_index("s")
  wid = lax.axis_index("c") * ns + sid
  s0 = wid * seg_w

  @pl.when(sid == 0)
  def _():
    pltpu.sync_copy(ctrl8, shtab)

  plsc.subcore_barrier()

  pltpu.sync_copy(cpad.at[pl.ds(s0, seg_w + 32)], cseg)

  def _cs(i):
    return cseg[pl.ds(i, _LANES)][0]

  p1 = _cs(seg_w)

  def refill_if(trigger, bs, wb):
    new_base = bs & ~jnp.int32(_ALIGN - 1)
    row0 = jnp.minimum(new_base // _IDXW, max_row0)
    row0 = pl.multiple_of(row0, 8)
    wb_new = jnp.where(trigger, row0 * _IDXW, wb)

    @pl.when(trigger)
    def _():
      cp_i = pltpu.async_copy(jm2.at[pl.ds(row0, _ROWS)], idx2d, sem_i)
      cp_t = pltpu.async_copy(tp.at[pl.ds(row0 * _IDXW, _CHUNK)], tpv, sem_t)
      cp_i.wait()

      def fire(j, _):
        pltpu.async_copy(shtab.at[idx2d.at[j]],
                         rows.at[pl.ds(j * _IDXW, _IDXW)], sem_g)
        return 0

      lax.fori_loop(0, _ROWS, fire, 0)
      cp_t.wait()
      pltpu.make_async_copy(ctrl8.at[idx2d.at[0]], rows, sem_g).wait()

    return wb_new

  def seg_body(s_, carry):
    wbase, cs = carry
    cn = _cs(s_ + 1)
    plen = cn - cs
    nfull = plen >> 4
    rem = plen & (_LANES - 1)

    def step(i, c):
      wb, vx, vy, vz = c
      bs = cs + i * _LANES
      wb = refill_if(bs + _LANES > wb + _CHUNK, bs, wb)
      off = (bs - wb) + iota
      tpg = plsc.load_gather(tpv, [off])
      gx = plsc.load_gather(rows, [off, col0])
      gy = plsc.load_gather(rows, [off, col1])
      gz = plsc.load_gather(rows, [off, col2])
      return (wb, vx + tpg * gx, vy + tpg * gy, vz + tpg * gz)

    wbase, vx, vy, vz = lax.fori_loop(0, nfull, step,
                                      (wbase, fzero, fzero, fzero))

    bs = cs + (nfull << 4)
    wbase = refill_if((rem > 0) & (jnp.minimum(bs + _LANES, p1)
                                   > wbase + _CHUNK), bs, wbase)
    valid = iota < rem
    off = (bs - wbase) + iota
    tpg = jnp.where(valid, plsc.load_gather(tpv, [off], mask=valid), 0.0)
    gx = jnp.where(valid, plsc.load_gather(rows, [off, col0], mask=valid), 0.0)
    gy = jnp.where(valid, plsc.load_gather(rows, [off, col1], mask=valid), 0.0)
    gz = jnp.where(valid, plsc.load_gather(rows, [off, col2], mask=valid), 0.0)
    vx = vx + tpg * gx
    vy = vy + tpg * gy
    vz = vz + tpg * gz

    sx = jnp.sum(vx)
    sy = jnp.sum(vy)
    sz = jnp.sum(vz)
    contrib = (jnp.where(iota == 0, sx, 0.0)
               + jnp.where(iota == 1, sy, 0.0)
               + jnp.where(iota == 2, sz, 0.0))
    plsc.store_scatter(acc, [4 * s_ + iota], contrib, mask=m3)
    return (wbase, cn)

  lax.fori_loop(0, seg_w, seg_body, (jnp.int32(-(2 ** 30)), _cs(0)))

  pltpu.sync_copy(acc.at[pl.ds(0, seg_w * 4)],
                  out.at[pl.ds(wid * seg_w * 4, seg_w * 4)])


def kernel(ctrl_pts, Jm_array, tensor_prod, num_supp_bs_cumsum):
  num_ctrl = ctrl_pts.shape[0]
  total_supp = Jm_array.shape[0]
  num_eval = num_supp_bs_cumsum.shape[0] - 1

  try:
    info = plsc.get_sparse_core_info()
    nc, ns = info.num_cores, info.num_subcores
  except ValueError:
    nc, ns = 2, 16
  nw = nc * ns
  seg_w = num_eval // nw
  assert num_eval % nw == 0 and total_supp % _CHUNK == 0

  ctrl8 = jnp.pad(ctrl_pts, ((0, 0), (0, 5)))
  jm2 = Jm_array.reshape(total_supp // _IDXW, _IDXW)
  cpad = jnp.pad(num_supp_bs_cumsum, (0, 32))

  accw = seg_w * 4 + _LANES

  mesh = plsc.VectorSubcoreMesh(core_axis_name="c", subcore_axis_name="s",
                                num_cores=nc, num_subcores=ns)
  out_flat = pl.kernel(
      functools.partial(_sc_body, ns, total_supp, seg_w),
      out_type=jax.ShapeDtypeStruct((num_eval * 4,), jnp.float32),
      mesh=mesh,
      compiler_params=pltpu.CompilerParams(needs_layout_passes=False,
                                           use_tc_tiling_on_sc=False),
      scratch_types=[
          pltpu.VMEM_SHARED((num_ctrl, 8), jnp.float32),
          pltpu.VMEM((seg_w + 32,), jnp.int32),
          pltpu.VMEM((_ROWS, _IDXW), jnp.int32),
          pltpu.VMEM((_CHUNK,), jnp.float32),
          pltpu.VMEM((_CHUNK, 8), jnp.float32),
          pltpu.VMEM((accw,), jnp.float32),
          pltpu.SemaphoreType.DMA,
          pltpu.SemaphoreType.DMA,
          pltpu.SemaphoreType.DMA,
      ],
  )(ctrl8, jm2, tensor_prod, cpad)

  return out_flat.reshape(num_eval, 4)[:, :3]


# speedup vs baseline: 1.2775x; 1.0512x over previous
import functools

import jax
import jax.numpy as jnp
from jax import lax
from jax.experimental import pallas as pl
from jax.experimental.pallas import tpu as pltpu
from jax.experimental.pallas import tpu_sc as plsc

_LANES = 16
_CHUNK = 3072
_STRIDE = 2048
_IDXW = 128
_ROWS = _CHUNK // _IDXW
_SROWS = _STRIDE // _IDXW


def _sc_body(ns, total_supp, seg_w,
             ctrl8, jm2, tp, cpad, out,
             shtab, cseg, idx2d, tpv, rows, acc,
             sem_i, sem_t, sem_g):
  iota = lax.iota(jnp.int32, _LANES)
  col1 = jnp.full((_LANES,), 1, jnp.int32)
  col2 = jnp.full((_LANES,), 2, jnp.int32)
  col0 = jnp.zeros((_LANES,), jnp.int32)
  m3 = iota < 3
  fzero = jnp.zeros((_LANES,), jnp.float32)
  max_row0 = total_supp // _IDXW - _ROWS

  sid = lax.axis_index("s")
  wid = lax.axis_index("c") * ns + sid
  s0 = wid * seg_w

  @pl.when(sid == 0)
  def _():
    pltpu.sync_copy(ctrl8, shtab)

  plsc.subcore_barrier()

  pltpu.sync_copy(cpad.at[pl.ds(s0, seg_w + 32)], cseg)

  def _cs(i):
    return cseg[pl.ds(i, _LANES)][0]

  p0 = _cs(0)
  p1 = _cs(seg_w)
  row00 = (p0 >> 7) & ~jnp.int32(7)

  def _rowof(k):
    r = jnp.minimum(row00 + k * _SROWS, max_row0)
    return pl.multiple_of(r, 8)

  def _wbof(k):
    return _rowof(k) * _IDXW

  def _fire_idx(k):
    par = k & 1
    pltpu.async_copy(jm2.at[pl.ds(_rowof(k), _ROWS)],
                     idx2d.at[pl.ds(par * _ROWS, _ROWS)], sem_i)

  def _fire_gtp(k):
    par = k & 1
    row = _rowof(k)
    pltpu.async_copy(tp.at[pl.ds(row * _IDXW, _CHUNK)],
                     tpv.at[pl.ds(par * _CHUNK, _CHUNK)], sem_t)

    def fire(j, _):
      pltpu.async_copy(shtab.at[idx2d.at[par * _ROWS + j]],
                       rows.at[pl.ds(par * _CHUNK + j * _IDXW, _IDXW)],
                       sem_g)
      return 0

    lax.fori_loop(0, _ROWS, fire, 0)

  def _drain_g():
    pltpu.make_async_copy(ctrl8.at[idx2d.at[0]],
                          rows.at[pl.ds(0, _CHUNK)], sem_g).wait()

  def _wait_tp():
    pltpu.make_async_copy(tp.at[pl.ds(0, _CHUNK)],
                          tpv.at[pl.ds(0, _CHUNK)], sem_t).wait()

  def _wait_idx():
    pltpu.make_async_copy(jm2.at[pl.ds(0, _ROWS)],
                          idx2d.at[pl.ds(0, _ROWS)], sem_i).wait()

  _fire_idx(0)
  _wait_idx()
  _fire_gtp(0)
  _fire_idx(1)
  _drain_g()
  _wait_tp()
  _wait_idx()
  _fire_gtp(2 - 1)
  _fire_idx(2)

  def _advance(trigger, k):
    @pl.when(trigger)
    def _():
      _drain_g()
      _wait_tp()
      _wait_idx()
      _fire_gtp(k + 2)
      _fire_idx(k + 3)

    return jnp.where(trigger, k + 1, k)

  def seg_body(s_, carry):
    k, cs = carry
    cn = _cs(s_ + 1)
    plen = cn - cs
    nfull = plen >> 4
    rem = plen & (_LANES - 1)

    def step(i, c):
      k_, vx, vy, vz = c
      bs = cs + i * _LANES
      k_ = _advance(bs + _LANES > _wbof(k_) + _CHUNK, k_)
      off = (bs - _wbof(k_)) + (k_ & 1) * _CHUNK + iota
      tpg = plsc.load_gather(tpv, [off])
      gx = plsc.load_gather(rows, [off, col0])
      gy = plsc.load_gather(rows, [off, col1])
      gz = plsc.load_gather(rows, [off, col2])
      return (k_, vx + tpg * gx, vy + tpg * gy, vz + tpg * gz)

    k, vx, vy, vz = lax.fori_loop(0, nfull, step, (k, fzero, fzero, fzero))

    bs = cs + (nfull << 4)
    k = _advance((rem > 0) & (jnp.minimum(bs + _LANES, p1)
                              > _wbof(k) + _CHUNK), k)
    valid = iota < rem
    off = (bs - _wbof(k)) + (k & 1) * _CHUNK + iota
    tpg = jnp.where(valid, plsc.load_gather(tpv, [off], mask=valid), 0.0)
    gx = jnp.where(valid, plsc.load_gather(rows, [off, col0], mask=valid), 0.0)
    gy = jnp.where(valid, plsc.load_gather(rows, [off, col1], mask=valid), 0.0)
    gz = jnp.where(valid, plsc.load_gather(rows, [off, col2], mask=valid), 0.0)
    vx = vx + tpg * gx
    vy = vy + tpg * gy
    vz = vz + tpg * gz

    sx = jnp.sum(vx)
    sy = jnp.sum(vy)
    sz = jnp.sum(vz)
    contrib = (jnp.where(iota == 0, sx, 0.0)
               + jnp.where(iota == 1, sy, 0.0)
               + jnp.where(iota == 2, sz, 0.0))
    plsc.store_scatter(acc, [4 * s_ + iota], contrib, mask=m3)
    return (k, cn)

  lax.fori_loop(0, seg_w, seg_body, (jnp.int32(0), p0))

  _drain_g()
  _wait_tp()
  _wait_idx()

  pltpu.sync_copy(acc.at[pl.ds(0, seg_w * 4)],
                  out.at[pl.ds(wid * seg_w * 4, seg_w * 4)])


def kernel(ctrl_pts, Jm_array, tensor_prod, num_supp_bs_cumsum):
  num_ctrl = ctrl_pts.shape[0]
  total_supp = Jm_array.shape[0]
  num_eval = num_supp_bs_cumsum.shape[0] - 1

  try:
    info = plsc.get_sparse_core_info()
    nc, ns = info.num_cores, info.num_subcores
  except ValueError:
    nc, ns = 2, 16
  nw = nc * ns
  seg_w = num_eval // nw
  assert num_eval % nw == 0 and total_supp % _IDXW == 0

  ctrl8 = jnp.pad(ctrl_pts, ((0, 0), (0, 5)))
  jm2 = Jm_array.reshape(total_supp // _IDXW, _IDXW)
  cpad = jnp.pad(num_supp_bs_cumsum, (0, 32))

  accw = seg_w * 4 + _LANES

  mesh = plsc.VectorSubcoreMesh(core_axis_name="c", subcore_axis_name="s",
                                num_cores=nc, num_subcores=ns)
  out_flat = pl.kernel(
      functools.partial(_sc_body, ns, total_supp, seg_w),
      out_type=jax.ShapeDtypeStruct((num_eval * 4,), jnp.float32),
      mesh=mesh,
      compiler_params=pltpu.CompilerParams(needs_layout_passes=False,
                                           use_tc_tiling_on_sc=False),
      scratch_types=[
          pltpu.VMEM_SHARED((num_ctrl, 8), jnp.float32),
          pltpu.VMEM((seg_w + 32,), jnp.int32),
          pltpu.VMEM((2 * _ROWS, _IDXW), jnp.int32),
          pltpu.VMEM((2 * _CHUNK,), jnp.float32),
          pltpu.VMEM((2 * _CHUNK, 8), jnp.float32),
          pltpu.VMEM((accw,), jnp.float32),
          pltpu.SemaphoreType.DMA,
          pltpu.SemaphoreType.DMA,
          pltpu.SemaphoreType.DMA,
      ],
  )(ctrl8, jm2, tensor_prod, cpad)

  return out_flat.reshape(num_eval, 4)[:, :3]


# speedup vs baseline: 1.5581x; 1.2197x over previous
import functools

import jax
import jax.numpy as jnp
from jax import lax
from jax.experimental import pallas as pl
from jax.experimental.pallas import tpu as pltpu
from jax.experimental.pallas import tpu_sc as plsc

_LANES = 16
_CHUNK = 3072
_STRIDE = 2048
_IDXW = 128
_ROWS = _CHUNK // _IDXW
_SROWS = _STRIDE // _IDXW


def _sc_body(ns, total_supp, seg_w,
             ctrl8, jm2, tp, cpad, out,
             shtab, cseg, idx2d, tpv, rows, acc,
             sem_i, sem_t, sem_g):
  iota = lax.iota(jnp.int32, _LANES)
  col1 = jnp.full((_LANES,), 1, jnp.int32)
  col2 = jnp.full((_LANES,), 2, jnp.int32)
  col0 = jnp.zeros((_LANES,), jnp.int32)
  m3 = iota < 3
  fzero = jnp.zeros((_LANES,), jnp.float32)
  max_row0 = total_supp // _IDXW - _ROWS

  sid = lax.axis_index("s")
  wid = lax.axis_index("c") * ns + sid
  s0 = wid * seg_w

  @pl.when(sid == 0)
  def _():
    pltpu.sync_copy(ctrl8, shtab)

  plsc.subcore_barrier()

  pltpu.sync_copy(cpad.at[pl.ds(s0, seg_w + 32)], cseg)

  def _cs(i):
    return cseg[pl.ds(i, _LANES)][0]

  p0 = _cs(0)
  p1 = _cs(seg_w)
  row00 = (p0 >> 7) & ~jnp.int32(7)

  def _rowof(k):
    r = jnp.minimum(row00 + k * _SROWS, max_row0)
    return pl.multiple_of(r, 8)

  def _wbof(k):
    return _rowof(k) * _IDXW

  def _fire_idx(k):
    par = k & 1
    pltpu.async_copy(jm2.at[pl.ds(_rowof(k), _ROWS)],
                     idx2d.at[pl.ds(par * _ROWS, _ROWS)], sem_i)

  def _fire_gtp(k):
    par = k & 1
    row = _rowof(k)
    pltpu.async_copy(tp.at[pl.ds(row * _IDXW, _CHUNK)],
                     tpv.at[pl.ds(par * _CHUNK, _CHUNK)], sem_t)

    def fire(j, _):
      pltpu.async_copy(shtab.at[idx2d.at[par * _ROWS + j]],
                       rows.at[pl.ds(par * _CHUNK + j * _IDXW, _IDXW)],
                       sem_g)
      return 0

    lax.fori_loop(0, _ROWS, fire, 0)

  def _drain_g():
    pltpu.make_async_copy(ctrl8.at[idx2d.at[0]],
                          rows.at[pl.ds(0, _CHUNK)], sem_g).wait()

  def _wait_tp():
    pltpu.make_async_copy(tp.at[pl.ds(0, _CHUNK)],
                          tpv.at[pl.ds(0, _CHUNK)], sem_t).wait()

  def _wait_idx():
    pltpu.make_async_copy(jm2.at[pl.ds(0, _ROWS)],
                          idx2d.at[pl.ds(0, _ROWS)], sem_i).wait()

  _fire_idx(0)
  _wait_idx()
  _fire_gtp(0)
  _fire_idx(1)
  _drain_g()
  _wait_tp()
  _wait_idx()
  _fire_gtp(2 - 1)
  _fire_idx(2)

  def _advance(trigger, k):
    @pl.when(trigger)
    def _():
      _drain_g()
      _wait_tp()
      _wait_idx()
      _fire_gtp(k + 2)
      _fire_idx(k + 3)

    return jnp.where(trigger, k + 1, k)

  def seg_body(s_, carry):
    k, cs = carry
    cn = _cs(s_ + 1)
    plen = cn - cs
    nfull = plen >> 4
    rem = plen & (_LANES - 1)
    npieces = (nfull + 63) >> 6

    def piece(p, c):
      k_, vx, vy, vz = c
      bs0 = cs + (p << 10)
      cnt = jnp.minimum(nfull - (p << 6), 64)
      k_ = _advance(bs0 + (cnt << 4) > _wbof(k_) + _CHUNK, k_)
      off0 = (bs0 - _wbof(k_)) + (k_ & 1) * _CHUNK + iota

      def blk(i, cc):
        vx_, vy_, vz_, off = cc
        tpg = plsc.load_gather(tpv, [off])
        gx = plsc.load_gather(rows, [off, col0])
        gy = plsc.load_gather(rows, [off, col1])
        gz = plsc.load_gather(rows, [off, col2])
        return (vx_ + tpg * gx, vy_ + tpg * gy, vz_ + tpg * gz,
                off + _LANES)

      vx, vy, vz, _ = lax.fori_loop(0, cnt, blk, (vx, vy, vz, off0))
      return (k_, vx, vy, vz)

    k, vx, vy, vz = lax.fori_loop(0, npieces, piece,
                                  (k, fzero, fzero, fzero))

    bs = cs + (nfull << 4)
    k = _advance((rem > 0) & (jnp.minimum(bs + _LANES, p1)
                              > _wbof(k) + _CHUNK), k)
    valid = iota < rem
    off = (bs - _wbof(k)) + (k & 1) * _CHUNK + iota
    tpg = jnp.where(valid, plsc.load_gather(tpv, [off], mask=valid), 0.0)
    gx = jnp.where(valid, plsc.load_gather(rows, [off, col0], mask=valid), 0.0)
    gy = jnp.where(valid, plsc.load_gather(rows, [off, col1], mask=valid), 0.0)
    gz = jnp.where(valid, plsc.load_gather(rows, [off, col2], mask=valid), 0.0)
    vx = vx + tpg * gx
    vy = vy + tpg * gy
    vz = vz + tpg * gz

    sx = jnp.sum(vx)
    sy = jnp.sum(vy)
    sz = jnp.sum(vz)
    contrib = (jnp.where(iota == 0, sx, 0.0)
               + jnp.where(iota == 1, sy, 0.0)
               + jnp.where(iota == 2, sz, 0.0))
    plsc.store_scatter(acc, [3 * s_ + iota], contrib, mask=m3)
    return (k, cn)

  lax.fori_loop(0, seg_w, seg_body, (jnp.int32(0), p0))

  _drain_g()
  _wait_tp()
  _wait_idx()

  pltpu.sync_copy(acc.at[pl.ds(0, seg_w * 3)],
                  out.at[pl.ds(wid * seg_w * 3, seg_w * 3)])


def kernel(ctrl_pts, Jm_array, tensor_prod, num_supp_bs_cumsum):
  num_ctrl = ctrl_pts.shape[0]
  total_supp = Jm_array.shape[0]
  num_eval = num_supp_bs_cumsum.shape[0] - 1

  try:
    info = plsc.get_sparse_core_info()
    nc, ns = info.num_cores, info.num_subcores
  except ValueError:
    nc, ns = 2, 16
  nw = nc * ns
  seg_w = num_eval // nw
  assert num_eval % nw == 0 and total_supp % _IDXW == 0

  ctrl8 = jnp.pad(ctrl_pts, ((0, 0), (0, 5)))
  jm2 = Jm_array.reshape(total_supp // _IDXW, _IDXW)
  cpad = jnp.pad(num_supp_bs_cumsum, (0, 32))

  accw = seg_w * 3 + _LANES

  mesh = plsc.VectorSubcoreMesh(core_axis_name="c", subcore_axis_name="s",
                                num_cores=nc, num_subcores=ns)
  out_flat = pl.kernel(
      functools.partial(_sc_body, ns, total_supp, seg_w),
      out_type=jax.ShapeDtypeStruct((num_eval * 3,), jnp.float32),
      mesh=mesh,
      compiler_params=pltpu.CompilerParams(needs_layout_passes=False,
                                           use_tc_tiling_on_sc=False),
      scratch_types=[
          pltpu.VMEM_SHARED((num_ctrl, 8), jnp.float32),
          pltpu.VMEM((seg_w + 32,), jnp.int32),
          pltpu.VMEM((2 * _ROWS, _IDXW), jnp.int32),
          pltpu.VMEM((2 * _CHUNK,), jnp.float32),
          pltpu.VMEM((2 * _CHUNK, 8), jnp.float32),
          pltpu.VMEM((accw,), jnp.float32),
          pltpu.SemaphoreType.DMA,
          pltpu.SemaphoreType.DMA,
          pltpu.SemaphoreType.DMA,
      ],
  )(ctrl8, jm2, tensor_prod, cpad)

  return out_flat.reshape(num_eval, 3)
